# baseline (speedup 1.0000x reference)
import functools

import jax
import jax.numpy as jnp
from jax import lax
from jax.experimental import pallas as pl
from jax.experimental.pallas import tpu as pltpu
from jax.experimental.pallas import tpu_sc as plsc

_B, _S, _H = 1, 2048, 1024
_E, _K, _I = 64, 2, 512
_T = _B * _S
_NP = _T * _K
_BM = 64
_P = 8192
_NB = _P // _BM
_TB = 256
_NTB = _T // _TB
_NW = 32
_DPW = _P // _NW
_TPW = _T // _NW


def _router_body(x_ref, gw_ref, sel0_ref, sel1_ref, rw0_ref, rw1_ref,
                 fcopy_ref):
    x = x_ref[...]
    fcopy_ref[...] = x
    gw = gw_ref[...]
    logits = lax.dot_general(x, gw, (((1,), (1,)), ((), ())),
                             preferred_element_type=jnp.float32)
    col = lax.broadcasted_iota(jnp.int32, logits.shape, 1)
    m1 = jnp.max(logits, axis=1, keepdims=True)
    a1 = jnp.min(jnp.where(logits == m1, col, _E), axis=1, keepdims=True)
    masked = jnp.where(col == a1, -jnp.inf, logits)
    m2 = jnp.max(masked, axis=1, keepdims=True)
    a2 = jnp.min(jnp.where(masked == m2, col, _E), axis=1, keepdims=True)
    e = jnp.exp(m2 - m1)
    r0 = 1.0 / (1.0 + e)
    sel0_ref[0, 0, :] = a1[:, 0]
    sel1_ref[0, 0, :] = a2[:, 0]
    rw0_ref[0, 0, :] = r0[:, 0]
    rw1_ref[0, 0, :] = (e * r0)[:, 0]


def _ffn_body(be_ref, x_ref, w1_ref, w2_ref, ws_ref, out_ref):
    del be_ref
    x = x_ref[...].astype(jnp.bfloat16)
    w1 = w1_ref[0].astype(jnp.bfloat16)
    gu = lax.dot_general(x, w1, (((1,), (1,)), ((), ())),
                         preferred_element_type=jnp.float32)
    g = gu[:, :_I]
    u = gu[:, _I:]
    act = (g * jax.nn.sigmoid(g) * u).astype(jnp.bfloat16)
    w2 = w2_ref[0].astype(jnp.bfloat16)
    out = lax.dot_general(act, w2, (((1,), (1,)), ((), ())),
                          preferred_element_type=jnp.float32)
    out_ref[...] = out * ws_ref[0, 0, :][:, None]


_CH = 32


def _dispatch_body(idx_hbm, src_hbm, out_hbm, idx_v, rows_a, rows_b,
                   sga, sgb, swa, swb):
    wid = lax.axis_index("s") * 2 + lax.axis_index("c")
    base = wid * _DPW
    pltpu.sync_copy(idx_hbm.at[pl.ds(base, _DPW)], idx_v)
    bufs = (rows_a, rows_b)
    gsem = (sga, sgb)
    wsem = (swa, swb)
    nc = _DPW // _CH
    g = [None] * nc
    w = [None] * nc
    for c in range(2):
        g[c] = pltpu.async_copy(
            src_hbm.at[idx_v.at[pl.ds(c * _CH, _CH)]], bufs[c], gsem[c])
    for c in range(nc):
        b = c & 1
        g[c].wait()
        w[c] = pltpu.async_copy(
            bufs[b], out_hbm.at[pl.ds(base + c * _CH, _CH)], wsem[b])
        if c + 2 < nc:
            w[c].wait()
            g[c + 2] = pltpu.async_copy(
                src_hbm.at[idx_v.at[pl.ds((c + 2) * _CH, _CH)]],
                bufs[b], gsem[b])
    w[nc - 2].wait()
    w[nc - 1].wait()


def _combine_body(p0_hbm, p1_hbm, osr_hbm, out_hbm, i0_v, i1_v, b0_v, b1_v,
                  sem):
    wid = lax.axis_index("s") * 2 + lax.axis_index("c")
    base = wid * _TPW
    pltpu.sync_copy(p0_hbm.at[pl.ds(base, _TPW)], i0_v)
    pltpu.sync_copy(p1_hbm.at[pl.ds(base, _TPW)], i1_v)
    for c in range(_TPW // 32):
        pltpu.async_copy(osr_hbm.at[i0_v.at[pl.ds(c * 32, 32)]],
                         b0_v, sem).wait()
        pltpu.async_copy(osr_hbm.at[i1_v.at[pl.ds(c * 32, 32)]],
                         b1_v, sem).wait()

        def row(r, carry):
            for j in range(_H // 16):
                sl = pl.ds(j * 16, 16)
                b0_v[r, sl] = b0_v[r, sl] + b1_v[r, sl]
            return carry

        lax.fori_loop(0, 32, row, None)
        pltpu.sync_copy(b0_v, out_hbm.at[pl.ds(base + c * 32, 32)])


def _routing_meta(sel0, sel1, rw0, rw1):
    sel = jnp.stack([sel0, sel1], axis=1)
    rwf = jnp.stack([rw0, rw1], axis=1).reshape(-1)
    ep = sel.reshape(-1)
    order = jnp.argsort(ep, stable=True).astype(jnp.int32)
    counts = jnp.zeros((_E,), jnp.int32).at[ep].add(1)
    offs = jnp.concatenate(
        [jnp.zeros((1,), jnp.int32), jnp.cumsum(counts)[:-1]])
    pc = ((counts + _BM - 1) // _BM) * _BM
    po = jnp.concatenate([jnp.zeros((1,), jnp.int32), jnp.cumsum(pc)[:-1]])
    es = ep[order]
    rank = jnp.arange(_NP, dtype=jnp.int32) - offs[es]
    pos_sorted = po[es] + rank
    row_token = jnp.zeros((_P,), jnp.int32).at[pos_sorted].set(
        (order // _K).astype(jnp.int32))
    w_sorted = jnp.zeros((_P,), jnp.float32).at[pos_sorted].set(rwf[order])
    block_expert = jnp.clip(
        jnp.searchsorted(po, jnp.arange(_NB, dtype=jnp.int32) * _BM,
                         side='right').astype(jnp.int32) - 1, 0, _E - 1)
    pos_pair = jnp.zeros((_NP,), jnp.int32).at[order].set(pos_sorted)
    return row_token, w_sorted, block_expert, pos_pair[0::2], pos_pair[1::2]


@jax.jit
def kernel(hidden_states, gate_w, w1_stacked, w2_stacked):
    flat = hidden_states.reshape(_T, _H)

    sel0, sel1, rw0, rw1, flat_sc = pl.pallas_call(
        _router_body,
        grid=(_NTB,),
        in_specs=[pl.BlockSpec((_TB, _H), lambda i: (i, 0)),
                  pl.BlockSpec((_E, _H), lambda i: (0, 0))],
        out_specs=[pl.BlockSpec((1, 1, _TB), lambda i: (i, 0, 0))] * 4
        + [pl.BlockSpec((_TB, _H), lambda i: (i, 0))],
        out_shape=[jax.ShapeDtypeStruct((_NTB, 1, _TB), jnp.int32),
                   jax.ShapeDtypeStruct((_NTB, 1, _TB), jnp.int32),
                   jax.ShapeDtypeStruct((_NTB, 1, _TB), jnp.float32),
                   jax.ShapeDtypeStruct((_NTB, 1, _TB), jnp.float32),
                   jax.ShapeDtypeStruct((_T, _H), jnp.float32)],
    )(flat, gate_w)

    row_token, w_sorted, block_expert, pos0, pos1 = _routing_meta(
        sel0.reshape(_T), sel1.reshape(_T),
        rw0.reshape(_T), rw1.reshape(_T))

    sc_mesh = plsc.VectorSubcoreMesh(core_axis_name="c", subcore_axis_name="s")

    dispatch = functools.partial(
        pl.kernel,
        out_type=jax.ShapeDtypeStruct((_P, _H), jnp.float32),
        mesh=sc_mesh,
        scratch_types=[pltpu.VMEM((_DPW,), jnp.int32),
                       pltpu.VMEM((_CH, _H), jnp.float32),
                       pltpu.VMEM((_CH, _H), jnp.float32),
                       pltpu.SemaphoreType.DMA,
                       pltpu.SemaphoreType.DMA,
                       pltpu.SemaphoreType.DMA,
                       pltpu.SemaphoreType.DMA],
    )(_dispatch_body)
    x_sorted = dispatch(row_token, flat_sc)

    grid_spec = pltpu.PrefetchScalarGridSpec(
        num_scalar_prefetch=1,
        grid=(_NB,),
        in_specs=[
            pl.BlockSpec((_BM, _H), lambda i, be: (i, 0)),
            pl.BlockSpec((1, 2 * _I, _H), lambda i, be: (be[i], 0, 0)),
            pl.BlockSpec((1, _H, _I), lambda i, be: (be[i], 0, 0)),
            pl.BlockSpec((1, 1, _BM), lambda i, be: (i, 0, 0)),
        ],
        out_specs=pl.BlockSpec((_BM, _H), lambda i, be: (i, 0)),
    )
    out_sorted = pl.pallas_call(
        _ffn_body,
        grid_spec=grid_spec,
        out_shape=jax.ShapeDtypeStruct((_P, _H), jnp.float32),
    )(block_expert, x_sorted, w1_stacked, w2_stacked,
      w_sorted.reshape(_NB, 1, _BM))

    combine = functools.partial(
        pl.kernel,
        out_type=jax.ShapeDtypeStruct((_T, _H), jnp.float32),
        mesh=sc_mesh,
        scratch_types=[pltpu.VMEM((_TPW,), jnp.int32),
                       pltpu.VMEM((_TPW,), jnp.int32),
                       pltpu.VMEM((32, _H), jnp.float32),
                       pltpu.VMEM((32, _H), jnp.float32),
                       pltpu.SemaphoreType.DMA],
    )(_combine_body)
    out = combine(pos0, pos1, out_sorted)

    return out.reshape(_B, _S, _H)


# speedup vs baseline: 1.1763x; 1.1763x over previous
import functools

import jax
import jax.numpy as jnp
from jax import lax
from jax.experimental import pallas as pl
from jax.experimental.pallas import tpu as pltpu
from jax.experimental.pallas import tpu_sc as plsc

_B, _S, _H = 1, 2048, 1024
_E, _K, _I = 64, 2, 512
_T = _B * _S
_NP = _T * _K
_BM = 64
_P = 8192
_NB = _P // _BM
_TB = 256
_NTB = _T // _TB
_NW = 32
_DPW = _P // _NW
_TPW = _T // _NW


def _router_body(x_ref, gw_ref, sel0_ref, sel1_ref, rw0_ref, rw1_ref):
    x = x_ref[...]
    gw = gw_ref[...]
    logits = lax.dot_general(x, gw, (((1,), (1,)), ((), ())),
                             preferred_element_type=jnp.float32)
    col = lax.broadcasted_iota(jnp.int32, logits.shape, 1)
    m1 = jnp.max(logits, axis=1, keepdims=True)
    a1 = jnp.min(jnp.where(logits == m1, col, _E), axis=1, keepdims=True)
    masked = jnp.where(col == a1, -jnp.inf, logits)
    m2 = jnp.max(masked, axis=1, keepdims=True)
    a2 = jnp.min(jnp.where(masked == m2, col, _E), axis=1, keepdims=True)
    e = jnp.exp(m2 - m1)
    r0 = 1.0 / (1.0 + e)
    sel0_ref[0, 0, :] = a1[:, 0]
    sel1_ref[0, 0, :] = a2[:, 0]
    rw0_ref[0, 0, :] = r0[:, 0]
    rw1_ref[0, 0, :] = (e * r0)[:, 0]


def _ffn_body(be_ref, x_ref, w1_ref, w2_ref, ws_ref, out_ref):
    del be_ref
    x = x_ref[...].astype(jnp.bfloat16)
    w1 = w1_ref[0].astype(jnp.bfloat16)
    gu = lax.dot_general(x, w1, (((1,), (1,)), ((), ())),
                         preferred_element_type=jnp.float32)
    g = gu[:, :_I]
    u = gu[:, _I:]
    act = (g * jax.nn.sigmoid(g) * u).astype(jnp.bfloat16)
    w2 = w2_ref[0].astype(jnp.bfloat16)
    out = lax.dot_general(act, w2, (((1,), (1,)), ((), ())),
                          preferred_element_type=jnp.float32)
    out_ref[...] = out * ws_ref[0, 0, :][:, None]


_CH = 32


def _dispatch_body(idx_hbm, src_hbm, out_hbm, idx_v, rows_a, rows_b,
                   sga, sgb, swa, swb):
    wid = lax.axis_index("s") * 2 + lax.axis_index("c")
    base = wid * _DPW
    pltpu.sync_copy(idx_hbm.at[pl.ds(base, _DPW)], idx_v)
    bufs = (rows_a, rows_b)
    gsem = (sga, sgb)
    wsem = (swa, swb)
    nc = _DPW // _CH
    g = [None] * nc
    w = [None] * nc
    for c in range(2):
        g[c] = pltpu.async_copy(
            src_hbm.at[idx_v.at[pl.ds(c * _CH, _CH)]], bufs[c], gsem[c])
    for c in range(nc):
        b = c & 1
        g[c].wait()
        w[c] = pltpu.async_copy(
            bufs[b], out_hbm.at[pl.ds(base + c * _CH, _CH)], wsem[b])
        if c + 2 < nc:
            w[c].wait()
            g[c + 2] = pltpu.async_copy(
                src_hbm.at[idx_v.at[pl.ds((c + 2) * _CH, _CH)]],
                bufs[b], gsem[b])
    w[nc - 2].wait()
    w[nc - 1].wait()


def _combine_body(p0_hbm, p1_hbm, osr_hbm, out_hbm, i0_v, i1_v, b0_v, b1_v,
                  sem):
    wid = lax.axis_index("s") * 2 + lax.axis_index("c")
    base = wid * _TPW
    pltpu.sync_copy(p0_hbm.at[pl.ds(base, _TPW)], i0_v)
    pltpu.sync_copy(p1_hbm.at[pl.ds(base, _TPW)], i1_v)
    for c in range(_TPW // 32):
        pltpu.async_copy(osr_hbm.at[i0_v.at[pl.ds(c * 32, 32)]],
                         b0_v, sem).wait()
        pltpu.async_copy(osr_hbm.at[i1_v.at[pl.ds(c * 32, 32)]],
                         b1_v, sem).wait()

        def row(r, carry):
            for j in range(_H // 16):
                sl = pl.ds(j * 16, 16)
                b0_v[r, sl] = b0_v[r, sl] + b1_v[r, sl]
            return carry

        lax.fori_loop(0, 32, row, None)
        pltpu.sync_copy(b0_v, out_hbm.at[pl.ds(base + c * 32, 32)])


def _routing_meta(sel0, sel1, rw0, rw1):
    sel = jnp.stack([sel0, sel1], axis=1)
    rwf = jnp.stack([rw0, rw1], axis=1).reshape(-1)
    ep = sel.reshape(-1)
    order = jnp.argsort(ep, stable=True).astype(jnp.int32)
    es = ep[order]
    ohf = (es[:, None] == jnp.arange(_E, dtype=jnp.int32)[None, :]
           ).astype(jnp.float32)
    counts = ohf.sum(axis=0).astype(jnp.int32)
    offs = jnp.concatenate(
        [jnp.zeros((1,), jnp.int32), jnp.cumsum(counts)[:-1]])
    pc = ((counts + _BM - 1) // _BM) * _BM
    po = jnp.concatenate([jnp.zeros((1,), jnp.int32), jnp.cumsum(pc)[:-1]])
    tabs = jnp.stack([offs, po], axis=1).astype(jnp.float32)
    lk = (ohf @ tabs).astype(jnp.int32)
    rank = jnp.arange(_NP, dtype=jnp.int32) - lk[:, 0]
    pos_sorted = lk[:, 1] + rank
    row_token = jnp.zeros((_P,), jnp.int32).at[pos_sorted].set(
        (order // _K).astype(jnp.int32))
    w_sorted = jnp.zeros((_P,), jnp.float32).at[pos_sorted].set(rwf[order])
    block_expert = (jnp.sum(
        po[None, :] <= jnp.arange(_NB, dtype=jnp.int32)[:, None] * _BM,
        axis=1).astype(jnp.int32) - 1).clip(0, _E - 1)
    pos_pair = jnp.zeros((_NP,), jnp.int32).at[order].set(pos_sorted)
    return row_token, w_sorted, block_expert, pos_pair[0::2], pos_pair[1::2]


@jax.jit
def kernel(hidden_states, gate_w, w1_stacked, w2_stacked):
    flat = hidden_states.reshape(_T, _H)

    sel0, sel1, rw0, rw1 = pl.pallas_call(
        _router_body,
        grid=(_NTB,),
        in_specs=[pl.BlockSpec((_TB, _H), lambda i: (i, 0)),
                  pl.BlockSpec((_E, _H), lambda i: (0, 0))],
        out_specs=[pl.BlockSpec((1, 1, _TB), lambda i: (i, 0, 0))] * 4,
        out_shape=[jax.ShapeDtypeStruct((_NTB, 1, _TB), jnp.int32),
                   jax.ShapeDtypeStruct((_NTB, 1, _TB), jnp.int32),
                   jax.ShapeDtypeStruct((_NTB, 1, _TB), jnp.float32),
                   jax.ShapeDtypeStruct((_NTB, 1, _TB), jnp.float32)],
    )(flat, gate_w)

    row_token, w_sorted, block_expert, pos0, pos1 = _routing_meta(
        sel0.reshape(_T), sel1.reshape(_T),
        rw0.reshape(_T), rw1.reshape(_T))

    sc_mesh = plsc.VectorSubcoreMesh(core_axis_name="c", subcore_axis_name="s")

    dispatch = functools.partial(
        pl.kernel,
        out_type=jax.ShapeDtypeStruct((_P, _H), jnp.float32),
        mesh=sc_mesh,
        scratch_types=[pltpu.VMEM((_DPW,), jnp.int32),
                       pltpu.VMEM((_CH, _H), jnp.float32),
                       pltpu.VMEM((_CH, _H), jnp.float32),
                       pltpu.SemaphoreType.DMA,
                       pltpu.SemaphoreType.DMA,
                       pltpu.SemaphoreType.DMA,
                       pltpu.SemaphoreType.DMA],
    )(_dispatch_body)
    x_sorted = dispatch(row_token, flat)

    grid_spec = pltpu.PrefetchScalarGridSpec(
        num_scalar_prefetch=1,
        grid=(_NB,),
        in_specs=[
            pl.BlockSpec((_BM, _H), lambda i, be: (i, 0)),
            pl.BlockSpec((1, 2 * _I, _H), lambda i, be: (be[i], 0, 0)),
            pl.BlockSpec((1, _H, _I), lambda i, be: (be[i], 0, 0)),
            pl.BlockSpec((1, 1, _BM), lambda i, be: (i, 0, 0)),
        ],
        out_specs=pl.BlockSpec((_BM, _H), lambda i, be: (i, 0)),
    )
    out_sorted = pl.pallas_call(
        _ffn_body,
        grid_spec=grid_spec,
        out_shape=jax.ShapeDtypeStruct((_P, _H), jnp.float32),
    )(block_expert, x_sorted, w1_stacked, w2_stacked,
      w_sorted.reshape(_NB, 1, _BM))

    combine = functools.partial(
        pl.kernel,
        out_type=jax.ShapeDtypeStruct((_T, _H), jnp.float32),
        mesh=sc_mesh,
        scratch_types=[pltpu.VMEM((_TPW,), jnp.int32),
                       pltpu.VMEM((_TPW,), jnp.int32),
                       pltpu.VMEM((32, _H), jnp.float32),
                       pltpu.VMEM((32, _H), jnp.float32),
                       pltpu.SemaphoreType.DMA],
    )(_combine_body)
    out = combine(pos0, pos1, out_sorted)

    return out.reshape(_B, _S, _H)


# speedup vs baseline: 1.1798x; 1.0030x over previous
import functools

import jax
import jax.numpy as jnp
from jax import lax
from jax.experimental import pallas as pl
from jax.experimental.pallas import tpu as pltpu
from jax.experimental.pallas import tpu_sc as plsc

_B, _S, _H = 1, 2048, 1024
_E, _K, _I = 64, 2, 512
_T = _B * _S
_NP = _T * _K
_BM = 64
_P = 8192
_NB = _P // _BM
_TB = 256
_NTB = _T // _TB
_NW = 32
_DPW = _P // _NW
_TPW = _T // _NW


def _router_body(x_ref, gw_ref, sel0_ref, sel1_ref, rw0_ref, rw1_ref):
    x = x_ref[...]
    gw = gw_ref[...]
    logits = lax.dot_general(x, gw, (((1,), (1,)), ((), ())),
                             preferred_element_type=jnp.float32)
    col = lax.broadcasted_iota(jnp.int32, logits.shape, 1)
    m1 = jnp.max(logits, axis=1, keepdims=True)
    a1 = jnp.min(jnp.where(logits == m1, col, _E), axis=1, keepdims=True)
    masked = jnp.where(col == a1, -jnp.inf, logits)
    m2 = jnp.max(masked, axis=1, keepdims=True)
    a2 = jnp.min(jnp.where(masked == m2, col, _E), axis=1, keepdims=True)
    e = jnp.exp(m2 - m1)
    r0 = 1.0 / (1.0 + e)
    sel0_ref[0, 0, :] = a1[:, 0]
    sel1_ref[0, 0, :] = a2[:, 0]
    rw0_ref[0, 0, :] = r0[:, 0]
    rw1_ref[0, 0, :] = (e * r0)[:, 0]


def _ffn_body(be_ref, x_ref, w1_ref, w2_ref, ws_ref, out_ref):
    del be_ref
    x = x_ref[...].astype(jnp.bfloat16)
    w1 = w1_ref[0].astype(jnp.bfloat16)
    gu = lax.dot_general(x, w1, (((1,), (1,)), ((), ())),
                         preferred_element_type=jnp.float32)
    g = gu[:, :_I]
    u = gu[:, _I:]
    act = (g * jax.nn.sigmoid(g) * u).astype(jnp.bfloat16)
    w2 = w2_ref[0].astype(jnp.bfloat16)
    out = lax.dot_general(act, w2, (((1,), (1,)), ((), ())),
                          preferred_element_type=jnp.float32)
    out_ref[...] = out * ws_ref[0, 0, :][:, None]


_CH = 32


def _dispatch_body(idx_hbm, src_hbm, out_hbm, idx_v, rows_a, rows_b,
                   sga, sgb, swa, swb):
    wid = lax.axis_index("s") * 2 + lax.axis_index("c")
    base = wid * _DPW
    pltpu.sync_copy(idx_hbm.at[pl.ds(base, _DPW)], idx_v)
    bufs = (rows_a, rows_b)
    gsem = (sga, sgb)
    wsem = (swa, swb)
    nc = _DPW // _CH
    g = [None] * nc
    w = [None] * nc
    for c in range(2):
        g[c] = pltpu.async_copy(
            src_hbm.at[idx_v.at[pl.ds(c * _CH, _CH)]], bufs[c], gsem[c])
    for c in range(nc):
        b = c & 1
        g[c].wait()
        w[c] = pltpu.async_copy(
            bufs[b], out_hbm.at[pl.ds(base + c * _CH, _CH)], wsem[b])
        if c + 2 < nc:
            w[c].wait()
            g[c + 2] = pltpu.async_copy(
                src_hbm.at[idx_v.at[pl.ds((c + 2) * _CH, _CH)]],
                bufs[b], gsem[b])
    w[nc - 2].wait()
    w[nc - 1].wait()


def _combine_body(p0_hbm, p1_hbm, osr_hbm, out_hbm, i0_v, i1_v, b0_v, b1_v,
                  sem):
    wid = lax.axis_index("s") * 2 + lax.axis_index("c")
    base = wid * _TPW
    pltpu.sync_copy(p0_hbm.at[pl.ds(base, _TPW)], i0_v)
    pltpu.sync_copy(p1_hbm.at[pl.ds(base, _TPW)], i1_v)
    for c in range(_TPW // 32):
        pltpu.async_copy(osr_hbm.at[i0_v.at[pl.ds(c * 32, 32)]],
                         b0_v, sem).wait()
        pltpu.async_copy(osr_hbm.at[i1_v.at[pl.ds(c * 32, 32)]],
                         b1_v, sem).wait()

        def row(r, carry):
            for j in range(_H // 16):
                sl = pl.ds(j * 16, 16)
                b0_v[r, sl] = b0_v[r, sl] + b1_v[r, sl]
            return carry

        lax.fori_loop(0, 32, row, None)
        pltpu.sync_copy(b0_v, out_hbm.at[pl.ds(base + c * 32, 32)])


def _routing_meta(sel0, sel1, rw0, rw1):
    sel = jnp.stack([sel0, sel1], axis=1)
    rwf = jnp.stack([rw0, rw1], axis=1).reshape(-1)
    ep = sel.reshape(-1)
    order = jnp.argsort(ep, stable=True).astype(jnp.int32)
    es = ep[order]
    ohf = (es[:, None] == jnp.arange(_E, dtype=jnp.int32)[None, :]
           ).astype(jnp.float32)
    counts = ohf.sum(axis=0).astype(jnp.int32)
    offs = jnp.concatenate(
        [jnp.zeros((1,), jnp.int32), jnp.cumsum(counts)[:-1]])
    pc = ((counts + _BM - 1) // _BM) * _BM
    po = jnp.concatenate([jnp.zeros((1,), jnp.int32), jnp.cumsum(pc)[:-1]])
    tabs = jnp.stack([offs, po], axis=1).astype(jnp.float32)
    lk = jnp.round(ohf @ tabs).astype(jnp.int32)
    rank = jnp.arange(_NP, dtype=jnp.int32) - lk[:, 0]
    pos_sorted = lk[:, 1] + rank
    row_token = jnp.zeros((_P,), jnp.int32).at[pos_sorted].set(
        (order // _K).astype(jnp.int32))
    w_sorted = jnp.zeros((_P,), jnp.float32).at[pos_sorted].set(rwf[order])
    block_expert = (jnp.sum(
        po[None, :] <= jnp.arange(_NB, dtype=jnp.int32)[:, None] * _BM,
        axis=1).astype(jnp.int32) - 1).clip(0, _E - 1)
    pos_pair = jnp.zeros((_NP,), jnp.int32).at[order].set(pos_sorted)
    return row_token, w_sorted, block_expert, pos_pair[0::2], pos_pair[1::2]


@jax.jit
def kernel(hidden_states, gate_w, w1_stacked, w2_stacked):
    flat = hidden_states.reshape(_T, _H)

    sel0, sel1, rw0, rw1 = pl.pallas_call(
        _router_body,
        grid=(_NTB,),
        in_specs=[pl.BlockSpec((_TB, _H), lambda i: (i, 0)),
                  pl.BlockSpec((_E, _H), lambda i: (0, 0))],
        out_specs=[pl.BlockSpec((1, 1, _TB), lambda i: (i, 0, 0))] * 4,
        out_shape=[jax.ShapeDtypeStruct((_NTB, 1, _TB), jnp.int32),
                   jax.ShapeDtypeStruct((_NTB, 1, _TB), jnp.int32),
                   jax.ShapeDtypeStruct((_NTB, 1, _TB), jnp.float32),
                   jax.ShapeDtypeStruct((_NTB, 1, _TB), jnp.float32)],
    )(flat, gate_w)

    row_token, w_sorted, block_expert, pos0, pos1 = _routing_meta(
        sel0.reshape(_T), sel1.reshape(_T),
        rw0.reshape(_T), rw1.reshape(_T))

    sc_mesh = plsc.VectorSubcoreMesh(core_axis_name="c", subcore_axis_name="s")

    dispatch = functools.partial(
        pl.kernel,
        out_type=jax.ShapeDtypeStruct((_P, _H), jnp.float32),
        mesh=sc_mesh,
        scratch_types=[pltpu.VMEM((_DPW,), jnp.int32),
                       pltpu.VMEM((_CH, _H), jnp.float32),
                       pltpu.VMEM((_CH, _H), jnp.float32),
                       pltpu.SemaphoreType.DMA,
                       pltpu.SemaphoreType.DMA,
                       pltpu.SemaphoreType.DMA,
                       pltpu.SemaphoreType.DMA],
    )(_dispatch_body)
    x_sorted = dispatch(row_token, flat)

    grid_spec = pltpu.PrefetchScalarGridSpec(
        num_scalar_prefetch=1,
        grid=(_NB,),
        in_specs=[
            pl.BlockSpec((_BM, _H), lambda i, be: (i, 0)),
            pl.BlockSpec((1, 2 * _I, _H), lambda i, be: (be[i], 0, 0)),
            pl.BlockSpec((1, _H, _I), lambda i, be: (be[i], 0, 0)),
            pl.BlockSpec((1, 1, _BM), lambda i, be: (i, 0, 0)),
        ],
        out_specs=pl.BlockSpec((_BM, _H), lambda i, be: (i, 0)),
    )
    out_sorted = pl.pallas_call(
        _ffn_body,
        grid_spec=grid_spec,
        out_shape=jax.ShapeDtypeStruct((_P, _H), jnp.float32),
    )(block_expert, x_sorted, w1_stacked, w2_stacked,
      w_sorted.reshape(_NB, 1, _BM))

    combine = functools.partial(
        pl.kernel,
        out_type=jax.ShapeDtypeStruct((_T, _H), jnp.float32),
        mesh=sc_mesh,
        scratch_types=[pltpu.VMEM((_TPW,), jnp.int32),
                       pltpu.VMEM((_TPW,), jnp.int32),
                       pltpu.VMEM((32, _H), jnp.float32),
                       pltpu.VMEM((32, _H), jnp.float32),
                       pltpu.SemaphoreType.DMA],
    )(_combine_body)
    out = combine(pos0, pos1, out_sorted)

    return out.reshape(_B, _S, _H)
